# baseline (speedup 1.0000x reference)
import functools

import jax
import jax.numpy as jnp
from jax import lax
from jax.experimental import pallas as pl
from jax.experimental.pallas import tpu as pltpu
from jax.experimental.pallas import tpu_sc as plsc

_VOCAB = 1000000
_DIM = 16
_BATCH = 16384

_NC = 2
_NS = 16
_NW = _NC * _NS
_BPW = _BATCH // _NW
_CHUNK = 128
_NCHUNK = _BPW // _CHUNK

_mesh = plsc.VectorSubcoreMesh(core_axis_name="c", subcore_axis_name="s")


@functools.partial(
    pl.kernel,
    mesh=_mesh,
    out_type=jax.ShapeDtypeStruct((_BATCH, _DIM), jnp.float32),
    scratch_types=[
        pltpu.VMEM((_NCHUNK, _CHUNK), jnp.int32),
        pltpu.VMEM((_BPW, _DIM), jnp.float32),
        pltpu.SemaphoreType.DMA,
    ],
    compiler_params=pltpu.CompilerParams(use_tc_tiling_on_sc=False),
)
def _gather_kernel(z_hbm, idx_hbm, out_hbm, idx_v, rows_v, sem):
    wid = lax.axis_index("s") * _NC + lax.axis_index("c")
    pltpu.sync_copy(idx_hbm.at[wid], idx_v)
    copies = []
    for j in range(_NCHUNK):
        copies.append(
            pltpu.async_copy(
                z_hbm.at[idx_v.at[j]],
                rows_v.at[pl.ds(j * _CHUNK, _CHUNK)],
                sem,
            )
        )
    for c in copies:
        c.wait()
    pltpu.sync_copy(rows_v, out_hbm.at[pl.ds(wid * _BPW, _BPW)])


def kernel(Z, indices):
    idx = indices.astype(jnp.int32).reshape(_NW, _NCHUNK, _CHUNK)
    return _gather_kernel(Z, idx)


# speedup vs baseline: 5.1819x; 5.1819x over previous
import functools

import jax
import jax.numpy as jnp
from jax import lax
from jax.experimental import pallas as pl
from jax.experimental.pallas import tpu as pltpu
from jax.experimental.pallas import tpu_sc as plsc

_VOCAB = 1000000
_DIM = 16
_BATCH = 16384

_NC = 2
_NS = 16
_NW = _NC * _NS
_BPW = _BATCH // _NW
_NBUF = 8

_mesh = plsc.VectorSubcoreMesh(core_axis_name="c", subcore_axis_name="s")


@functools.partial(
    pl.kernel,
    mesh=_mesh,
    out_type=jax.ShapeDtypeStruct((_BATCH, 128), jnp.float32),
    scratch_types=[
        pltpu.VMEM((_BPW + 16,), jnp.int32),
        pltpu.VMEM((_NBUF, _DIM, 128), jnp.float32),
        pltpu.VMEM((_BPW, 128), jnp.float32),
        [pltpu.SemaphoreType.DMA] * _NBUF,
    ],
    compiler_params=pltpu.CompilerParams(needs_layout_passes=False),
)
def _gather_kernel(zt_hbm, idx_hbm, out_hbm, idx_v, win_v, rows_v, sems):
    wid = lax.axis_index("s") * _NC + lax.axis_index("c")
    base = wid * _BPW
    pltpu.sync_copy(idx_hbm.at[pl.ds(base, _BPW)], idx_v.at[pl.ds(0, _BPW)])

    row_ids = lax.iota(jnp.int32, 16)

    def fire(j, slot):
        v = idx_v[pl.ds(j, 16)][0]
        col = pl.multiple_of((v // 128) * 128, 128)
        pltpu.async_copy(
            zt_hbm.at[:, pl.ds(col, 128)],
            win_v.at[slot],
            sems[slot],
        )

    for b in range(_NBUF):
        fire(b, b)

    def group(g, _):
        for b in range(_NBUF):
            j = g * _NBUF + b
            pltpu.make_async_copy(
                zt_hbm.at[:, pl.ds(0, 128)], win_v.at[b], sems[b]
            ).wait()
            v = idx_v[pl.ds(j, 16)][0]
            vl = lax.rem(v, 128)
            row = plsc.load_gather(
                win_v.at[b], [row_ids, jnp.full((16,), vl, jnp.int32)]
            )
            rows_v[j, pl.ds(0, _DIM)] = row

            @pl.when(j + _NBUF < _BPW)
            def _():
                fire(j + _NBUF, b)

        return ()

    lax.fori_loop(0, _BPW // _NBUF, group, ())
    pltpu.sync_copy(rows_v, out_hbm.at[pl.ds(base, _BPW), :])


def kernel(Z, indices):
    idx = indices.astype(jnp.int32)
    blob = _gather_kernel(Z.T, idx)
    return blob[:, :_DIM]
